# baseline (speedup 1.0000x reference)
import functools

import jax
import jax.numpy as jnp
from jax import lax
from jax.experimental import pallas as pl
from jax.experimental.pallas import tpu as pltpu
from jax.experimental.pallas import tpu_sc as plsc

NUM_CORES = 2
NUM_SUBCORES = 16
NUM_WORKERS = NUM_CORES * NUM_SUBCORES

CHUNK = 128
GPS = 2
NBUF = 2
SLOT = CHUNK * GPS


@functools.partial(jax.jit, static_argnames=("n_rows", "dim", "n_chunks"))
def _sc_gather(idx, weight, *, n_rows, dim, n_chunks):
    mesh = plsc.VectorSubcoreMesh(core_axis_name="c", subcore_axis_name="s")
    n_slots = n_chunks // GPS
    n_groups = n_slots // NBUF

    @functools.partial(
        pl.kernel,
        out_type=jax.ShapeDtypeStruct((n_rows, dim), jnp.float32),
        mesh=mesh,
        scratch_types=[
            pltpu.VMEM((n_chunks, CHUNK), jnp.int32),
            pltpu.VMEM((NBUF, SLOT, dim), jnp.float32),
            [pltpu.SemaphoreType.DMA] * NBUF,
            [pltpu.SemaphoreType.DMA] * NBUF,
        ],
        compiler_params=pltpu.CompilerParams(use_tc_tiling_on_sc=True),
    )
    def k(idx_hbm, table_hbm, out_hbm, idx_v, rows_v, gsems, osems):
        wid = lax.axis_index("s") * NUM_CORES + lax.axis_index("c")
        base = wid * (n_chunks * CHUNK)

        pltpu.sync_copy(idx_hbm.at[wid], idx_v)

        def gather(s, b):
            for u in range(GPS):
                pltpu.async_copy(
                    table_hbm.at[idx_v.at[s * GPS + u]],
                    rows_v.at[b].at[pl.ds(u * CHUNK, CHUNK)],
                    gsems[b],
                )

        def wait_gather(b):
            for u in range(GPS):
                pltpu.make_async_copy(
                    table_hbm.at[idx_v.at[0]],
                    rows_v.at[b].at[pl.ds(u * CHUNK, CHUNK)],
                    gsems[b],
                ).wait()

        def put(s, b):
            pltpu.async_copy(
                rows_v.at[b], out_hbm.at[pl.ds(base + s * SLOT, SLOT)], osems[b]
            )

        def wait_put(b):
            pltpu.make_async_copy(
                rows_v.at[b], out_hbm.at[pl.ds(base, SLOT)], osems[b]
            ).wait()

        for b in range(NBUF):
            gather(b, b)

        def group_body(g, carry):
            for b in range(NBUF):
                wait_gather(b)
                put(g * NBUF + b, b)
            for b in range(NBUF):
                wait_put(b)
                gather((g + 1) * NBUF + b, b)
            return carry

        lax.fori_loop(0, n_groups - 1, group_body, 0, unroll=False)

        for b in range(NBUF):
            wait_gather(b)
            put((n_groups - 1) * NBUF + b, b)
        for b in range(NBUF):
            wait_put(b)

    return k(idx, weight)


def kernel(input, weight):
    b, h = input.shape
    v, d = weight.shape
    n_rows = b * h
    per_worker = n_rows // NUM_WORKERS
    n_chunks = per_worker // CHUNK
    idx = input.T.reshape(NUM_WORKERS, n_chunks, CHUNK).astype(jnp.int32)
    out = _sc_gather(idx, weight, n_rows=n_rows, dim=d, n_chunks=n_chunks)
    return out.reshape(h, b, d).transpose(1, 0, 2)


# speedup vs baseline: 1.0077x; 1.0077x over previous
import functools

import jax
import jax.numpy as jnp
from jax import lax
from jax.experimental import pallas as pl
from jax.experimental.pallas import tpu as pltpu
from jax.experimental.pallas import tpu_sc as plsc

NUM_CORES = 2
NUM_SUBCORES = 16
NUM_WORKERS = NUM_CORES * NUM_SUBCORES

CHUNK = 128
GPS = 1
NBUF = 5
SLOT = CHUNK * GPS


@functools.partial(jax.jit, static_argnames=("n_rows", "dim", "n_chunks"))
def _sc_gather(idx, weight, *, n_rows, dim, n_chunks):
    mesh = plsc.VectorSubcoreMesh(core_axis_name="c", subcore_axis_name="s")
    n_slots = n_chunks // GPS
    n_groups = n_slots // NBUF

    @functools.partial(
        pl.kernel,
        out_type=jax.ShapeDtypeStruct((n_rows, dim), jnp.float32),
        mesh=mesh,
        scratch_types=[
            pltpu.VMEM((n_chunks, CHUNK), jnp.int32),
            pltpu.VMEM((NBUF, SLOT, dim), jnp.float32),
            [pltpu.SemaphoreType.DMA] * NBUF,
            [pltpu.SemaphoreType.DMA] * NBUF,
        ],
        compiler_params=pltpu.CompilerParams(use_tc_tiling_on_sc=True),
    )
    def k(idx_hbm, table_hbm, out_hbm, idx_v, rows_v, gsems, osems):
        wid = lax.axis_index("s") * NUM_CORES + lax.axis_index("c")
        base = wid * (n_chunks * CHUNK)

        pltpu.sync_copy(idx_hbm.at[wid], idx_v)

        def gather(s, b):
            for u in range(GPS):
                pltpu.async_copy(
                    table_hbm.at[idx_v.at[s * GPS + u]],
                    rows_v.at[b].at[pl.ds(u * CHUNK, CHUNK)],
                    gsems[b],
                )

        def wait_gather(b):
            for u in range(GPS):
                pltpu.make_async_copy(
                    table_hbm.at[idx_v.at[0]],
                    rows_v.at[b].at[pl.ds(u * CHUNK, CHUNK)],
                    gsems[b],
                ).wait()

        def put(s, b):
            pltpu.async_copy(
                rows_v.at[b], out_hbm.at[pl.ds(base + s * SLOT, SLOT)], osems[b]
            )

        def wait_put(b):
            pltpu.make_async_copy(
                rows_v.at[b], out_hbm.at[pl.ds(base, SLOT)], osems[b]
            ).wait()

        for b in range(NBUF):
            gather(b, b)

        def group_body(g, carry):
            for b in range(NBUF):
                wait_gather(b)
                put(g * NBUF + b, b)
            for b in range(NBUF):
                wait_put(b)
                gather((g + 1) * NBUF + b, b)
            return carry

        lax.fori_loop(0, n_groups - 1, group_body, 0, unroll=False)

        for b in range(NBUF):
            wait_gather(b)
            put((n_groups - 1) * NBUF + b, b)
        for b in range(NBUF):
            wait_put(b)

    return k(idx, weight)


def kernel(input, weight):
    b, h = input.shape
    v, d = weight.shape
    n_rows = b * h
    per_worker = n_rows // NUM_WORKERS
    n_chunks = per_worker // CHUNK
    idx = input.T.reshape(NUM_WORKERS, n_chunks, CHUNK).astype(jnp.int32)
    out = _sc_gather(idx, weight, n_rows=n_rows, dim=d, n_chunks=n_chunks)
    return out.reshape(h, b, d).transpose(1, 0, 2)


# speedup vs baseline: 1.0147x; 1.0070x over previous
import functools

import jax
import jax.numpy as jnp
from jax import lax
from jax.experimental import pallas as pl
from jax.experimental.pallas import tpu as pltpu
from jax.experimental.pallas import tpu_sc as plsc

NUM_CORES = 2
NUM_SUBCORES = 16
NUM_WORKERS = NUM_CORES * NUM_SUBCORES

IDX_ROW = 128
CHUNK = 64
SUB = IDX_ROW // CHUNK
NBUF = 10


@functools.partial(jax.jit, static_argnames=("n_rows", "dim", "n_idx_rows"))
def _sc_gather(idx, weight, *, n_rows, dim, n_idx_rows):
    mesh = plsc.VectorSubcoreMesh(core_axis_name="c", subcore_axis_name="s")
    n_slots = n_idx_rows * SUB
    n_groups = n_slots // NBUF

    @functools.partial(
        pl.kernel,
        out_type=jax.ShapeDtypeStruct((n_rows, dim), jnp.float32),
        mesh=mesh,
        scratch_types=[
            pltpu.VMEM((n_idx_rows, IDX_ROW), jnp.int32),
            pltpu.VMEM((NBUF, CHUNK, dim), jnp.float32),
            [pltpu.SemaphoreType.DMA] * NBUF,
            [pltpu.SemaphoreType.DMA] * NBUF,
        ],
        compiler_params=pltpu.CompilerParams(use_tc_tiling_on_sc=True),
    )
    def k(idx_hbm, table_hbm, out_hbm, idx_v, rows_v, gsems, osems):
        wid = lax.axis_index("s") * NUM_CORES + lax.axis_index("c")
        base = wid * (n_idx_rows * IDX_ROW)

        pltpu.sync_copy(idx_hbm.at[wid], idx_v)

        def gather(s, b):
            pltpu.async_copy(
                table_hbm.at[idx_v.at[s // SUB].at[pl.ds((s % SUB) * CHUNK, CHUNK)]],
                rows_v.at[b],
                gsems[b],
            )

        def wait_gather(b):
            pltpu.make_async_copy(
                table_hbm.at[idx_v.at[0].at[pl.ds(0, CHUNK)]],
                rows_v.at[b],
                gsems[b],
            ).wait()

        def put(s, b):
            pltpu.async_copy(
                rows_v.at[b], out_hbm.at[pl.ds(base + s * CHUNK, CHUNK)], osems[b]
            )

        def wait_put(b):
            pltpu.make_async_copy(
                rows_v.at[b], out_hbm.at[pl.ds(base, CHUNK)], osems[b]
            ).wait()

        for b in range(NBUF):
            gather(b, b)

        def group_body(g, carry):
            for b in range(NBUF):
                wait_gather(b)
                put(g * NBUF + b, b)
            for b in range(NBUF):
                wait_put(b)
                gather((g + 1) * NBUF + b, b)
            return carry

        lax.fori_loop(0, n_groups - 1, group_body, 0, unroll=False)

        for b in range(NBUF):
            wait_gather(b)
            put((n_groups - 1) * NBUF + b, b)
        for b in range(NBUF):
            wait_put(b)

    return k(idx, weight)


def kernel(input, weight):
    b, h = input.shape
    v, d = weight.shape
    n_rows = b * h
    per_worker = n_rows // NUM_WORKERS
    n_idx_rows = per_worker // IDX_ROW
    idx = input.T.reshape(NUM_WORKERS, n_idx_rows, IDX_ROW).astype(jnp.int32)
    out = _sc_gather(idx, weight, n_rows=n_rows, dim=d, n_idx_rows=n_idx_rows)
    return out.reshape(h, b, d).transpose(1, 0, 2)
